# baseline (speedup 1.0000x reference)
import functools

import jax
import jax.numpy as jnp
from jax import lax
from jax.experimental import pallas as pl
from jax.experimental.pallas import tpu as pltpu
from jax.experimental.pallas import tpu_sc as plsc

_VOCAB = 1000000
_DIM = 64
_MAX_LEN = 200
_BATCH = 4096
_SEQ = 200

_NC = 2
_NS = 16
_NW = _NC * _NS
_BPW = _BATCH // _NW
_KD = _DIM // 8
_NBUF = 4
_PF = 2


def _sinusoidal_pe():
    pos = jnp.arange(_MAX_LEN, dtype=jnp.float32)[:, None]
    div = jnp.exp(
        jnp.arange(0, _DIM, 2, dtype=jnp.float32) * (-jnp.log(10000.0) / _DIM)
    )
    pe = jnp.zeros((_MAX_LEN, _DIM), dtype=jnp.float32)
    pe = pe.at[:, 0::2].set(jnp.sin(pos * div))
    pe = pe.at[:, 1::2].set(jnp.cos(pos * div))
    return pe


@functools.partial(
    pl.kernel,
    mesh=plsc.VectorSubcoreMesh(core_axis_name="c", subcore_axis_name="s"),
    out_type=jax.ShapeDtypeStruct((_SEQ, _KD, _NW, 8, _BPW), jnp.float32),
    scratch_types=[
        pltpu.VMEM((_SEQ, _BPW), jnp.int32),
        pltpu.VMEM((_MAX_LEN, _DIM), jnp.float32),
        [pltpu.VMEM((_BPW, _DIM), jnp.float32) for _ in range(_NBUF)],
        [pltpu.VMEM((_KD, 8, _BPW), jnp.float32) for _ in range(_NBUF)],
        [pltpu.SemaphoreType.DMA for _ in range(_NBUF)],
        [pltpu.SemaphoreType.DMA for _ in range(_NBUF)],
    ],
    compiler_params=pltpu.CompilerParams(
        use_tc_tiling_on_sc=False, needs_layout_passes=False
    ),
)
def _emb_pe_sc(table_hbm, xt_hbm, pe_hbm, out_hbm, xt_v, pe_v, gbuf, tbuf,
               sg, ss):
    wid = lax.axis_index("s") * _NC + lax.axis_index("c")
    pltpu.sync_copy(pe_hbm, pe_v)
    pltpu.sync_copy(xt_hbm.at[:, pl.ds(wid * _BPW, _BPW)], xt_v)

    rows_g = [lax.iota(jnp.int32, 16) + 16 * g for g in range(_BPW // 16)]

    def gather(l, b):
        pltpu.async_copy(table_hbm.at[xt_v.at[l]], gbuf[b], sg[b])

    def gather_wait(l, b):
        pltpu.make_async_copy(table_hbm.at[xt_v.at[l]], gbuf[b], sg[b]).wait()

    def store(l, b):
        pltpu.async_copy(tbuf[b], out_hbm.at[l, :, wid], ss[b])

    def store_wait(l, b):
        pltpu.make_async_copy(tbuf[b], out_hbm.at[l, :, wid], ss[b]).wait()

    def transpose_add(l, b):
        lsplat = jnp.full((16,), l, jnp.int32)

        @plsc.parallel_loop(0, _DIM, step=1, unroll=2)
        def _(c):
            col = jnp.full((16,), c, jnp.int32)
            pe_c = plsc.load_gather(pe_v, [lsplat, col])
            k = c // 8
            s = c % 8
            for g in range(_BPW // 16):
                v = plsc.load_gather(gbuf[b], [rows_g[g], col])
                tbuf[b][k, s, pl.ds(16 * g, 16)] = v + pe_c

    for b in range(_PF):
        gather(b, b)

    def round_body(r, carry):
        for b in range(_NBUF):
            l = r * _NBUF + b
            j = l + _PF
            bp = (b + _PF) % _NBUF

            @pl.when(j < _SEQ)
            def _():
                gather(j, bp)

            gather_wait(l, b)

            @pl.when(l >= _NBUF)
            def _():
                store_wait(l, b)

            transpose_add(l, b)
            store(l, b)
        return carry

    lax.fori_loop(0, _SEQ // _NBUF, round_body, 0)
    for b in range(_NBUF):
        store_wait(0, b)


def kernel(x, table):
    pe = _sinusoidal_pe()
    xt = x.T.astype(jnp.int32)
    out5 = _emb_pe_sc(table, xt, pe)
    return out5.transpose(2, 4, 0, 1, 3).reshape(_BATCH, _SEQ, _DIM)


# speedup vs baseline: 1.3928x; 1.3928x over previous
import functools

import jax
import jax.numpy as jnp
from jax import lax
from jax.experimental import pallas as pl
from jax.experimental.pallas import tpu as pltpu
from jax.experimental.pallas import tpu_sc as plsc

_VOCAB = 1000000
_DIM = 64
_DPAD = 128
_MAX_LEN = 200
_BATCH = 4096
_SEQ = 200

_NC = 2
_NS = 16
_NW = _NC * _NS
_SPW = _BATCH // _NW
_ROWS = _BATCH * _SEQ
_RPW = _ROWS // _NW
_DV = _DIM // 16
_NBUF = 4
_PF = 2


def _sinusoidal_pe():
    pos = jnp.arange(_MAX_LEN, dtype=jnp.float32)[:, None]
    div = jnp.exp(
        jnp.arange(0, _DIM, 2, dtype=jnp.float32) * (-jnp.log(10000.0) / _DIM)
    )
    pe = jnp.zeros((_MAX_LEN, _DIM), dtype=jnp.float32)
    pe = pe.at[:, 0::2].set(jnp.sin(pos * div))
    pe = pe.at[:, 1::2].set(jnp.cos(pos * div))
    return pe


@functools.partial(
    pl.kernel,
    mesh=plsc.VectorSubcoreMesh(core_axis_name="c", subcore_axis_name="s"),
    out_type=jax.ShapeDtypeStruct((_ROWS, _DPAD), jnp.float32),
    scratch_types=[
        pltpu.VMEM((_SPW, _SEQ), jnp.int32),
        [pltpu.VMEM((_SEQ, _DIM), jnp.float32) for _ in range(_NBUF)],
        pltpu.VMEM((_MAX_LEN, _DIM), jnp.float32),
        [pltpu.SemaphoreType.DMA for _ in range(_NBUF)],
        [pltpu.SemaphoreType.DMA for _ in range(_NBUF)],
    ],
    compiler_params=pltpu.CompilerParams(use_tc_tiling_on_sc=False),
)
def _emb_pe_sc(table_hbm, x_hbm, pe_hbm, out_hbm, idx_v, rows, pe_v, sg, ss):
    wid = lax.axis_index("s") * _NC + lax.axis_index("c")
    base = wid * _SPW
    pltpu.sync_copy(pe_hbm, pe_v)
    pltpu.sync_copy(x_hbm.at[pl.ds(base, _SPW)], idx_v)

    def gather(j, b):
        pltpu.async_copy(table_hbm.at[idx_v.at[j]], rows[b], sg[b])

    def gather_wait(j, b):
        pltpu.make_async_copy(table_hbm.at[idx_v.at[j]], rows[b], sg[b]).wait()

    def store(i, b):
        dst = out_hbm.at[pl.ds(wid * _RPW + i * _SEQ, _SEQ), pl.ds(0, _DIM)]
        pltpu.async_copy(rows[b], dst, ss[b])

    def store_wait(b):
        dst = out_hbm.at[pl.ds(wid * _RPW, _SEQ), pl.ds(0, _DIM)]
        pltpu.make_async_copy(rows[b], dst, ss[b]).wait()

    def add_pe(b):
        @plsc.parallel_loop(0, _SEQ, step=1, unroll=8)
        def _(r):
            for d in range(_DV):
                sl = pl.ds(d * 16, 16)
                rows[b][r, sl] = rows[b][r, sl] + pe_v[r, sl]

    def step(i, b):
        j = i + _PF
        bp = (b + _PF) % _NBUF

        @pl.when(j < _SPW)
        def _():
            @pl.when(j >= _NBUF)
            def _():
                store_wait(bp)
            gather(j, bp)

        gather_wait(i, b)
        add_pe(b)
        store(i, b)

    for b in range(_PF):
        gather(b, b)

    def round_body(r, carry):
        for b in range(_NBUF):
            step(r * _NBUF + b, b)
        return carry

    lax.fori_loop(0, _SPW // _NBUF, round_body, 0)

    for b in range(_NBUF):
        store_wait(b)


def kernel(x, table):
    pe = _sinusoidal_pe()
    big = _emb_pe_sc(table, x.astype(jnp.int32), pe)
    return big[:, :_DIM].reshape(_BATCH, _SEQ, _DIM)
